# baseline (speedup 1.0000x reference)
import functools

import jax
import jax.numpy as jnp
from jax import lax
from jax.experimental import pallas as pl
from jax.experimental.pallas import tpu as pltpu
from jax.experimental.pallas import tpu_sc as plsc

EPS = 1e-5

_TC_BLOCK = 512


def _ln_table_body(table_ref, w_ref, b_ref, out_ref):
    t = table_ref[...]
    mean = jnp.mean(t, axis=-1, keepdims=True)
    var = jnp.mean(jnp.square(t - mean), axis=-1, keepdims=True)
    out_ref[...] = (t - mean) * lax.rsqrt(var + EPS) * w_ref[...] + b_ref[...]


def _normalize_table(table, ln_weight, ln_bias):
    rows, hidden = table.shape
    return pl.pallas_call(
        _ln_table_body,
        out_shape=jax.ShapeDtypeStruct((rows, hidden), jnp.float32),
    )(table, ln_weight.reshape(1, hidden), ln_bias.reshape(1, hidden))


def _select_body(ids_ref, tab_ref, out_ref):
    ids = ids_ref[0]
    m0 = ids == 0
    m1 = ids == 1
    r0 = tab_ref[0, :][None, :]
    r1 = tab_ref[1, :][None, :]
    r2 = tab_ref[2, :][None, :]
    out_ref[...] = jnp.where(m0, r0, jnp.where(m1, r1, r2))


def _tc_select(normed, idx, tokens, hidden, tb):
    nblk = tokens // tb
    rows = normed.shape[0]
    return pl.pallas_call(
        _select_body,
        grid=(nblk,),
        in_specs=[
            pl.BlockSpec((1, tb, 1), lambda i: (i, 0, 0)),
            pl.BlockSpec((rows, hidden), lambda i: (0, 0)),
        ],
        out_specs=pl.BlockSpec((tb, hidden), lambda i: (i, 0)),
        out_shape=jax.ShapeDtypeStruct((tokens, hidden), jnp.float32),
    )(idx.reshape(nblk, tb, 1), normed)


def _make_sc_gather(tokens, hidden, rows):
    info = plsc.get_sparse_core_info()
    nc, ns, nl = info.num_cores, info.num_subcores, info.num_lanes
    nw = nc * ns
    per_w = tokens // nw
    ngroups = per_w // nl
    mesh = plsc.VectorSubcoreMesh(core_axis_name="c", subcore_axis_name="s")

    @functools.partial(
        pl.kernel,
        mesh=mesh,
        out_type=jax.ShapeDtypeStruct((tokens, hidden), jnp.float32),
        scratch_types=[
            pltpu.VMEM((rows, hidden), jnp.float32),
            pltpu.VMEM((per_w,), jnp.int32),
            pltpu.VMEM((nl, hidden), jnp.float32),
            pltpu.SemaphoreType.DMA,
        ],
    )
    def sc_gather(normed_hbm, idx_hbm, out_hbm, tab_v, idx_v, drain_v, osem):
        wid = lax.axis_index("s") * nc + lax.axis_index("c")
        base = wid * per_w
        pltpu.sync_copy(normed_hbm, tab_v)
        pltpu.sync_copy(idx_hbm.at[pl.ds(base, per_w)], idx_v)

        def group(g, carry):
            iv = idx_v[pl.ds(g * nl, nl)]
            for j in range(nl):
                t = g * nl + j
                pltpu.async_copy(
                    tab_v.at[pl.ds(iv[j], 1)],
                    out_hbm.at[pl.ds(base + t, 1)],
                    osem,
                )
            @pl.when(g > 0)
            def _():
                pltpu.make_async_copy(
                    out_hbm.at[pl.ds(base, nl)], drain_v, osem
                ).wait()
            return carry

        lax.fori_loop(0, ngroups, group, 0)
        pltpu.make_async_copy(out_hbm.at[pl.ds(base, nl)], drain_v, osem).wait()

    return sc_gather


def kernel(type_token, table, ln_weight, ln_bias):
    b, s = type_token.shape
    rows, hidden = table.shape
    tokens = b * s
    normed = _normalize_table(table, ln_weight, ln_bias)
    idx = type_token.reshape(tokens).astype(jnp.int32)
    tc_tokens = (2 * tokens // 5) // _TC_BLOCK * _TC_BLOCK
    sc_tokens = tokens - tc_tokens
    sc_out = _make_sc_gather(sc_tokens, hidden, rows)(
        normed, lax.slice(idx, (tc_tokens,), (tokens,))
    )
    tc_out = _tc_select(
        normed, lax.slice(idx, (0,), (tc_tokens,)), tc_tokens, hidden, _TC_BLOCK
    )
    out = jnp.concatenate([tc_out, sc_out], axis=0)
    return out.reshape(b, s, hidden)


# speedup vs baseline: 2.3789x; 2.3789x over previous
import functools

import jax
import jax.numpy as jnp
from jax import lax
from jax.experimental import pallas as pl
from jax.experimental.pallas import tpu as pltpu
from jax.experimental.pallas import tpu_sc as plsc

EPS = 1e-5


def _ln_table_body(table_ref, w_ref, b_ref, out_ref):
    t = table_ref[...]
    mean = jnp.mean(t, axis=-1, keepdims=True)
    var = jnp.mean(jnp.square(t - mean), axis=-1, keepdims=True)
    out_ref[...] = (t - mean) * lax.rsqrt(var + EPS) * w_ref[...] + b_ref[...]


def _normalize_table(table, ln_weight, ln_bias):
    rows, hidden = table.shape
    return pl.pallas_call(
        _ln_table_body,
        out_shape=jax.ShapeDtypeStruct((rows, hidden), jnp.float32),
    )(table, ln_weight.reshape(1, hidden), ln_bias.reshape(1, hidden))


def _make_sc_gather(tokens, hidden, rows):
    info = plsc.get_sparse_core_info()
    nc, ns, nl = info.num_cores, info.num_subcores, info.num_lanes
    nw = nc * ns
    per_w = tokens // nw
    ngroups = per_w // nl
    mesh = plsc.VectorSubcoreMesh(core_axis_name="c", subcore_axis_name="s")

    @functools.partial(
        pl.kernel,
        mesh=mesh,
        out_type=jax.ShapeDtypeStruct((tokens, hidden), jnp.float32),
        scratch_types=[
            pltpu.VMEM((rows, hidden), jnp.float32),
            pltpu.VMEM((per_w,), jnp.int32),
            pltpu.VMEM((nl, hidden), jnp.float32),
            pltpu.SemaphoreType.DMA,
        ],
    )
    def sc_gather(normed_hbm, idx_hbm, out_hbm, tab_v, idx_v, drain_v, osem):
        wid = lax.axis_index("s") * nc + lax.axis_index("c")
        base = wid * per_w
        pltpu.sync_copy(normed_hbm, tab_v)
        pltpu.sync_copy(idx_hbm.at[pl.ds(base, per_w)], idx_v)

        def group(g, carry):
            iv = idx_v[pl.ds(g * nl, nl)]
            for j in range(nl):
                t = g * nl + j
                pltpu.async_copy(
                    tab_v.at[pl.ds(iv[j], 1)],
                    out_hbm.at[pl.ds(base + t, 1)],
                    osem,
                )
            @pl.when(g > 0)
            def _():
                pltpu.make_async_copy(
                    out_hbm.at[pl.ds(base, nl)], drain_v, osem
                ).wait()
            return carry

        lax.fori_loop(0, ngroups, group, 0)
        pltpu.make_async_copy(out_hbm.at[pl.ds(base, nl)], drain_v, osem).wait()

    return sc_gather


def kernel(type_token, table, ln_weight, ln_bias):
    b, s = type_token.shape
    rows, hidden = table.shape
    tokens = b * s
    normed = _normalize_table(table, ln_weight, ln_bias)
    idx = type_token.reshape(tokens).astype(jnp.int32)
    out = _make_sc_gather(tokens, hidden, rows)(normed, idx)
    return out.reshape(b, s, hidden)
